# baseline (speedup 1.0000x reference)
import jax
import jax.numpy as jnp
from jax.experimental import pallas as pl
from jax.experimental.pallas import tpu as pltpu


def _round_up(v, m):
    return (v + m - 1) // m * m


def _fused_kernel(adj_ref, x_ref, w1_ref, b1_ref, w2_ref, b2_ref,
                  wo_ref, bo_ref, out_ref, adj16_ref, h1_ref, h2lin_ref):
    p = pl.program_id(0)
    i = pl.program_id(1)
    tile_m = adj_ref.shape[0]
    rows = pl.ds(i * tile_m, tile_m)

    @pl.when(p == 0)
    def _phase1():
        a = adj_ref[...]
        adj16_ref[rows, :] = a.astype(jnp.bfloat16)
        agg = jnp.dot(a, x_ref[...], preferred_element_type=jnp.float32)
        rs = jnp.sum(a, axis=1, keepdims=True)
        pre1 = jnp.dot(agg, w1_ref[...], preferred_element_type=jnp.float32)
        h1 = jnp.maximum(pre1 + rs * b1_ref[...], 0.0)
        h1_ref[rows, :] = h1.astype(jnp.bfloat16)
        h2lin = jnp.dot(h1, w2_ref[...],
                        preferred_element_type=jnp.float32) + b2_ref[...]
        h2lin_ref[rows, :] = h2lin.astype(jnp.bfloat16)

    @pl.when(p == 1)
    def _phase2():
        h2 = jnp.maximum(
            jnp.dot(adj16_ref[rows, :], h2lin_ref[...],
                    preferred_element_type=jnp.float32), 0.0)
        logits = jnp.dot(h1_ref[rows, :].astype(jnp.float32) + h2, wo_ref[...],
                         preferred_element_type=jnp.float32)
        out_ref[...] = logits + bo_ref[...]


def _largest_tile(limit, size):
    t = limit
    while size % t:
        t //= 2
    return t


def kernel(x, adj, w1, b1, w2, b2, wo, bo):
    n, c = x.shape
    h = w1.shape[1]
    k_out = wo.shape[1]

    lane = 128
    c_pad = _round_up(c, lane)
    h_pad = _round_up(h, lane)
    k_pad = _round_up(k_out, lane)
    n_pad = _round_up(n, lane)

    tile_m = _largest_tile(128, n_pad)

    f32 = jnp.float32

    def _pad2(a, r, cc):
        a = a.astype(f32)
        if a.shape == (r, cc):
            return a
        return jnp.zeros((r, cc), f32).at[:a.shape[0], :a.shape[1]].set(a)

    x_p = _pad2(x, n_pad, c_pad)
    adj_p = _pad2(adj, n_pad, n_pad)
    w1_p = _pad2(w1, c_pad, h_pad)
    b1_p = _pad2(b1.reshape(1, -1), 1, h_pad)
    w2_p = _pad2(w2, h_pad, h_pad)
    b2_p = _pad2(b2.reshape(1, -1), 1, h_pad)
    wo_p = _pad2(wo, h_pad, k_pad)
    bo_p = _pad2(bo.reshape(1, -1), 1, k_pad)

    grid = (2, n_pad // tile_m)
    cparams = pltpu.CompilerParams(
        dimension_semantics=("arbitrary", "arbitrary"),
        vmem_limit_bytes=64 * 1024 * 1024,
    )

    logits_p = pl.pallas_call(
        _fused_kernel,
        out_shape=jax.ShapeDtypeStruct((n_pad, k_pad), jnp.float32),
        grid_spec=pltpu.PrefetchScalarGridSpec(
            num_scalar_prefetch=0,
            grid=grid,
            in_specs=[
                pl.BlockSpec((tile_m, n_pad), lambda p, i: (i * (1 - p), 0)),
                pl.BlockSpec((n_pad, c_pad), lambda p, i: (0, 0)),
                pl.BlockSpec((c_pad, h_pad), lambda p, i: (0, 0)),
                pl.BlockSpec((1, h_pad), lambda p, i: (0, 0)),
                pl.BlockSpec((h_pad, h_pad), lambda p, i: (0, 0)),
                pl.BlockSpec((1, h_pad), lambda p, i: (0, 0)),
                pl.BlockSpec((h_pad, k_pad), lambda p, i: (0, 0)),
                pl.BlockSpec((1, k_pad), lambda p, i: (0, 0)),
            ],
            out_specs=pl.BlockSpec((tile_m, k_pad), lambda p, i: (i * p, 0)),
            scratch_shapes=[
                pltpu.VMEM((n_pad, n_pad), jnp.bfloat16),
                pltpu.VMEM((n_pad, h_pad), jnp.bfloat16),
                pltpu.VMEM((n_pad, h_pad), jnp.bfloat16),
            ],
        ),
        compiler_params=cparams,
    )(adj_p, x_p, w1_p, b1_p, w2_p, b2_p, wo_p, bo_p)

    return logits_p[:n, :k_out]


# speedup vs baseline: 1.0926x; 1.0926x over previous
import jax
import jax.numpy as jnp
from jax.experimental import pallas as pl
from jax.experimental.pallas import tpu as pltpu


def _round_up(v, m):
    return (v + m - 1) // m * m


def _pass1_kernel(adj_ref, x_ref, w1_ref, b1_ref, w2_ref, b2_ref,
                  h1_ref, h2lin_ref, adj16_ref):
    a = adj_ref[...]
    adj16_ref[...] = a.astype(jnp.bfloat16)
    agg = jnp.dot(a, x_ref[...], preferred_element_type=jnp.float32)
    rs = jnp.sum(a, axis=1, keepdims=True)
    pre1 = jnp.dot(agg, w1_ref[...], preferred_element_type=jnp.float32)
    h1 = jnp.maximum(pre1 + rs * b1_ref[...], 0.0)
    h1_ref[...] = h1.astype(jnp.bfloat16)
    h2lin = jnp.dot(h1, w2_ref[...],
                    preferred_element_type=jnp.float32) + b2_ref[...]
    h2lin_ref[...] = h2lin.astype(jnp.bfloat16)


def _pass2_kernel(adj16_ref, h2lin_ref, h1_ref, wo_ref, bo_ref, out_ref):
    h2 = jnp.maximum(
        jnp.dot(adj16_ref[...], h2lin_ref[...],
                preferred_element_type=jnp.float32), 0.0)
    logits = jnp.dot(h1_ref[...].astype(jnp.float32) + h2, wo_ref[...],
                     preferred_element_type=jnp.float32)
    out_ref[...] = logits + bo_ref[...]


def _largest_tile(limit, size):
    t = limit
    while size % t:
        t //= 2
    return t


def kernel(x, adj, w1, b1, w2, b2, wo, bo):
    n, c = x.shape
    h = w1.shape[1]
    k_out = wo.shape[1]

    lane = 128
    c_pad = _round_up(c, lane)
    h_pad = _round_up(h, lane)
    k_pad = _round_up(k_out, lane)
    n_pad = _round_up(n, lane)

    tile_m = _largest_tile(1024, n_pad)

    f32 = jnp.float32
    bf = jnp.bfloat16

    def _pad2(a, r, cc):
        a = a.astype(f32)
        if a.shape == (r, cc):
            return a
        return jnp.zeros((r, cc), f32).at[:a.shape[0], :a.shape[1]].set(a)

    x_p = _pad2(x, n_pad, c_pad)
    adj_p = _pad2(adj, n_pad, n_pad)
    w1_p = _pad2(w1, c_pad, h_pad)
    b1_p = _pad2(b1.reshape(1, -1), 1, h_pad)
    w2_p = _pad2(w2, h_pad, h_pad)
    b2_p = _pad2(b2.reshape(1, -1), 1, h_pad)
    wo_p = _pad2(wo, h_pad, k_pad)
    bo_p = _pad2(bo.reshape(1, -1), 1, k_pad)

    grid = (n_pad // tile_m,)
    cparams = pltpu.CompilerParams(
        dimension_semantics=("arbitrary",),
        vmem_limit_bytes=64 * 1024 * 1024,
    )

    h1_p, h2lin_p, adj16 = pl.pallas_call(
        _pass1_kernel,
        out_shape=(jax.ShapeDtypeStruct((n_pad, h_pad), bf),
                   jax.ShapeDtypeStruct((n_pad, h_pad), bf),
                   jax.ShapeDtypeStruct((n_pad, n_pad), bf)),
        grid_spec=pltpu.PrefetchScalarGridSpec(
            num_scalar_prefetch=0,
            grid=grid,
            in_specs=[
                pl.BlockSpec((tile_m, n_pad), lambda i: (i, 0)),
                pl.BlockSpec((n_pad, c_pad), lambda i: (0, 0)),
                pl.BlockSpec((c_pad, h_pad), lambda i: (0, 0)),
                pl.BlockSpec((1, h_pad), lambda i: (0, 0)),
                pl.BlockSpec((h_pad, h_pad), lambda i: (0, 0)),
                pl.BlockSpec((1, h_pad), lambda i: (0, 0)),
            ],
            out_specs=(
                pl.BlockSpec((tile_m, h_pad), lambda i: (i, 0)),
                pl.BlockSpec((tile_m, h_pad), lambda i: (i, 0)),
                pl.BlockSpec((tile_m, n_pad), lambda i: (i, 0)),
            ),
        ),
        compiler_params=cparams,
    )(adj_p, x_p, w1_p, b1_p, w2_p, b2_p)

    logits_p = pl.pallas_call(
        _pass2_kernel,
        out_shape=jax.ShapeDtypeStruct((n_pad, k_pad), jnp.float32),
        grid_spec=pltpu.PrefetchScalarGridSpec(
            num_scalar_prefetch=0,
            grid=grid,
            in_specs=[
                pl.BlockSpec((tile_m, n_pad), lambda i: (i, 0)),
                pl.BlockSpec((n_pad, h_pad), lambda i: (0, 0)),
                pl.BlockSpec((tile_m, h_pad), lambda i: (i, 0)),
                pl.BlockSpec((h_pad, k_pad), lambda i: (0, 0)),
                pl.BlockSpec((1, k_pad), lambda i: (0, 0)),
            ],
            out_specs=pl.BlockSpec((tile_m, k_pad), lambda i: (i, 0)),
        ),
        compiler_params=cparams,
    )(adj16, h2lin_p, h1_p, wo_p, bo_p)

    return logits_p[:n, :k_out]


# speedup vs baseline: 1.4305x; 1.3092x over previous
import jax
import jax.numpy as jnp
from jax.experimental import pallas as pl
from jax.experimental.pallas import tpu as pltpu


def _round_up(v, m):
    return (v + m - 1) // m * m


def _fused_kernel(adj_ref, x_ref, w1_ref, b1_ref, w2_ref, b2_ref,
                  wo_ref, bo_ref, out_ref, h1_ref, h2lin_ref):
    p = pl.program_id(0)
    i = pl.program_id(1)
    tile_m = adj_ref.shape[0]
    rows = pl.ds(i * tile_m, tile_m)

    @pl.when(p == 0)
    def _phase1():
        a = adj_ref[...]
        agg = jnp.dot(a, x_ref[...], preferred_element_type=jnp.float32)
        rs = jnp.sum(a, axis=1, keepdims=True)
        pre1 = jnp.dot(agg, w1_ref[...], preferred_element_type=jnp.float32)
        h1 = jnp.maximum(pre1 + rs * b1_ref[...], 0.0)
        h1_ref[rows, :] = h1
        h2lin_ref[rows, :] = jnp.dot(
            h1, w2_ref[...], preferred_element_type=jnp.float32) + b2_ref[...]

    @pl.when(p == 1)
    def _phase2():
        h2 = jnp.maximum(
            jnp.dot(adj_ref[...], h2lin_ref[...],
                    preferred_element_type=jnp.float32), 0.0)
        logits = jnp.dot(h1_ref[rows, :] + h2, wo_ref[...],
                         preferred_element_type=jnp.float32)
        out_ref[...] = logits + bo_ref[...]


def _largest_tile(limit, size):
    t = limit
    while size % t:
        t //= 2
    return t


def kernel(x, adj, w1, b1, w2, b2, wo, bo):
    n, c = x.shape
    h = w1.shape[1]
    k_out = wo.shape[1]

    lane = 128
    c_pad = _round_up(c, lane)
    h_pad = _round_up(h, lane)
    k_pad = _round_up(k_out, lane)
    n_pad = _round_up(n, lane)

    tile_m = _largest_tile(1024, n_pad)

    f32 = jnp.float32

    def _pad2(a, r, cc):
        a = a.astype(f32)
        if a.shape == (r, cc):
            return a
        return jnp.zeros((r, cc), f32).at[:a.shape[0], :a.shape[1]].set(a)

    x_p = _pad2(x, n_pad, c_pad)
    adj_p = _pad2(adj, n_pad, n_pad)
    w1_p = _pad2(w1, c_pad, h_pad)
    b1_p = _pad2(b1.reshape(1, -1), 1, h_pad)
    w2_p = _pad2(w2, h_pad, h_pad)
    b2_p = _pad2(b2.reshape(1, -1), 1, h_pad)
    wo_p = _pad2(wo, h_pad, k_pad)
    bo_p = _pad2(bo.reshape(1, -1), 1, k_pad)

    grid = (2, n_pad // tile_m)
    cparams = pltpu.CompilerParams(
        dimension_semantics=("arbitrary", "arbitrary"),
        vmem_limit_bytes=64 * 1024 * 1024,
    )

    logits_p = pl.pallas_call(
        _fused_kernel,
        out_shape=jax.ShapeDtypeStruct((n_pad, k_pad), jnp.float32),
        grid_spec=pltpu.PrefetchScalarGridSpec(
            num_scalar_prefetch=0,
            grid=grid,
            in_specs=[
                pl.BlockSpec((tile_m, n_pad), lambda p, i: (i, 0)),
                pl.BlockSpec((n_pad, c_pad), lambda p, i: (0, 0)),
                pl.BlockSpec((c_pad, h_pad), lambda p, i: (0, 0)),
                pl.BlockSpec((1, h_pad), lambda p, i: (0, 0)),
                pl.BlockSpec((h_pad, h_pad), lambda p, i: (0, 0)),
                pl.BlockSpec((1, h_pad), lambda p, i: (0, 0)),
                pl.BlockSpec((h_pad, k_pad), lambda p, i: (0, 0)),
                pl.BlockSpec((1, k_pad), lambda p, i: (0, 0)),
            ],
            out_specs=pl.BlockSpec((tile_m, k_pad), lambda p, i: (i * p, 0)),
            scratch_shapes=[pltpu.VMEM((n_pad, h_pad), jnp.float32),
                            pltpu.VMEM((n_pad, h_pad), jnp.float32)],
        ),
        compiler_params=cparams,
    )(adj_p, x_p, w1_p, b1_p, w2_p, b2_p, wo_p, bo_p)

    return logits_p[:n, :k_out]
